# baseline (speedup 1.0000x reference)
import dataclasses
import functools

import jax
import jax.numpy as jnp
from jax import lax
from jax.experimental import pallas as pl
from jax.experimental.pallas import tpu as pltpu
from jax.experimental.pallas import tpu_sc as plsc

NUM_GRAPHS = 1024
OUT_DIM = 1024
NUM_WORKERS = 32
GMAX = 64
CHUNK = 8192
XBUF = CHUNK + 24
PTR_PAD = 1040
SPLIT_PAD = 48
LANES = 16
UNROLL = 4
HIST_WORDS = GMAX * OUT_DIM + 2048

_cp = pltpu.CompilerParams()
if "needs_layout_passes" in pltpu.CompilerParams.__dataclass_fields__:
    _cp = dataclasses.replace(_cp, needs_layout_passes=False)


@functools.partial(
    pl.kernel,
    compiler_params=_cp,
    out_type=jax.ShapeDtypeStruct((NUM_GRAPHS, OUT_DIM), jnp.float32),
    mesh=plsc.VectorSubcoreMesh(core_axis_name="c", subcore_axis_name="s"),
    scratch_types=[
        pltpu.VMEM((PTR_PAD,), jnp.int32),
        pltpu.VMEM((SPLIT_PAD,), jnp.int32),
        pltpu.VMEM((XBUF,), jnp.int32),
        pltpu.VMEM((XBUF,), jnp.int32),
        pltpu.VMEM((HIST_WORDS,), jnp.float32),
        pltpu.SemaphoreType.DMA,
        pltpu.SemaphoreType.DMA,
        pltpu.SemaphoreType.DMA,
    ],
)
def _count_kernel(x_hbm, ptr_hbm, gsplit_hbm, out_hbm, ptr_v, gsplit_v,
                  buf0, buf1, hist, sem0, sem1, wsem):
    total = x_hbm.shape[0]
    wid = lax.axis_index("s") * 2 + lax.axis_index("c")

    zeros16 = jnp.zeros((LANES,), jnp.float32)
    ones16 = jnp.ones((LANES,), jnp.float32)
    iota16 = lax.iota(jnp.int32, LANES)
    udim = jnp.uint32(OUT_DIM)

    pltpu.sync_copy(ptr_hbm, ptr_v)
    pltpu.sync_copy(gsplit_hbm, gsplit_v)

    gv = gsplit_v[pl.ds(wid, LANES)]
    gs = gv[0]
    ge = gv[1]
    ng = ge - gs

    @plsc.parallel_loop(0, ng * OUT_DIM, step=LANES, unroll=8)
    def _(i):
        hist[pl.ds(i, LANES)] = zeros16

    pw = ptr_v[pl.ds(gs, LANES)]
    wstart = pw[0]
    pe = ptr_v[pl.ds(ge, LANES)]
    wend = pe[0]
    wn = wend - wstart
    base_a = (wstart // 8) * 8
    nch = (wn + (CHUNK - 1)) // CHUNK

    def dma_start(c, buf, sem):
        a = jnp.minimum(base_a + c * CHUNK, total - (CHUNK + 8))
        pltpu.async_copy(x_hbm.at[pl.ds(a, CHUNK + 8)],
                         buf.at[pl.ds(0, CHUNK + 8)], sem)

    def dma_wait(buf, sem):
        pltpu.make_async_copy(x_hbm.at[pl.ds(0, CHUNK + 8)],
                              buf.at[pl.ds(0, CHUNK + 8)], sem).wait()

    def scat(colors, mask, bidx):
        idx = bidx + colors
        plsc.addupdate_scatter(hist, [idx], ones16, mask=mask)

    def process(c, buf, g):
        cs = wstart + c * CHUNK
        a = jnp.minimum(base_a + c * CHUNK, total - (CHUNK + 8))
        off = cs - a
        npc = jnp.minimum(wn - c * CHUNK, CHUNK)
        ce = cs + npc

        def piece_cond(st):
            p, _ = st
            return p < ce

        def piece(st):
            p, g = st
            pv = ptr_v[pl.ds(gs + g + 1, LANES)]
            gend = pv[0]
            e = jnp.minimum(gend, ce)
            n = e - p
            bidx = g * OUT_DIM
            boff = off + (p - cs)
            nfull = n // LANES

            @plsc.parallel_loop(0, nfull, unroll=UNROLL)
            def _(v):
                colors = buf[pl.ds(boff + v * LANES, LANES)]
                mask = plsc.bitcast(colors, jnp.uint32) < udim
                scat(colors, mask, bidx)

            rem = n - nfull * LANES

            @pl.when(rem > 0)
            def _():
                colors = buf[pl.ds(boff + nfull * LANES, LANES)]
                mask = (plsc.bitcast(colors, jnp.uint32) < udim) & (iota16 < rem)
                scat(colors, mask, bidx)

            g = jnp.where(gend <= ce, g + 1, g)
            return (e, g)

        _, g = lax.while_loop(piece_cond, piece, (cs, g))
        return g

    @pl.when(nch > 0)
    def _():
        dma_start(jnp.int32(0), buf0, sem0)

    def pair(i, g):
        c = 2 * i

        @pl.when(c + 1 < nch)
        def _():
            dma_start(c + 1, buf1, sem1)

        dma_wait(buf0, sem0)
        g = process(c, buf0, g)

        def second(g):
            @pl.when(c + 2 < nch)
            def _():
                dma_start(c + 2, buf0, sem0)

            dma_wait(buf1, sem1)
            return process(c + 1, buf1, g)

        return lax.cond(c + 1 < nch, second, lambda g: g, g)

    lax.fori_loop(0, (nch + 1) // 2, pair, jnp.int32(0))

    @pl.loop(0, ng)
    def _(g):
        pltpu.async_copy(hist.at[pl.ds(g * OUT_DIM, OUT_DIM)],
                         out_hbm.at[gs + g], wsem)

    @pl.loop(0, ng)
    def _(g):
        pltpu.make_async_copy(hist.at[pl.ds(0, OUT_DIM)],
                              out_hbm.at[0], wsem).wait()


def _balanced_splits(ptr32, total):
    w = jnp.arange(NUM_WORKERS + 1, dtype=jnp.int32)
    targets = (w * total) // NUM_WORKERS
    g = jnp.searchsorted(ptr32, targets, side="left").astype(jnp.int32)
    gm = jnp.clip(g - 1, 0, NUM_GRAPHS)
    gc = jnp.clip(g, 0, NUM_GRAPHS)
    pick = jnp.where(jnp.abs(ptr32[gm] - targets) <= jnp.abs(ptr32[gc] - targets),
                     gm, gc)
    pick = pick.at[0].set(0).at[-1].set(NUM_GRAPHS)
    pick = jax.lax.associative_scan(jnp.maximum, pick)
    low = jnp.maximum(0, NUM_GRAPHS - GMAX * (NUM_WORKERS - w))
    high = jnp.minimum(NUM_GRAPHS, GMAX * w)
    pick = jnp.clip(pick, low, high)

    def step(carry, p):
        v = jnp.minimum(carry + GMAX, p)
        return v, v

    _, capped = jax.lax.scan(step, pick[0], pick[1:])
    return jnp.concatenate([pick[:1], capped])


def kernel(x, ptr):
    x32 = x.astype(jnp.int32)
    ptr32 = ptr.astype(jnp.int32)
    total = x32.shape[0]
    gsplit = _balanced_splits(ptr32, total)
    gsplit = jnp.concatenate(
        [gsplit, jnp.full((SPLIT_PAD - gsplit.shape[0],), NUM_GRAPHS, jnp.int32)])
    ptrp = jnp.concatenate(
        [ptr32, jnp.full((PTR_PAD - ptr32.shape[0],), total, jnp.int32)])
    return _count_kernel(x32, ptrp, gsplit)


# speedup vs baseline: 2.1596x; 2.1596x over previous
import dataclasses
import functools

import jax
import jax.numpy as jnp
from jax import lax
from jax.experimental import pallas as pl
from jax.experimental.pallas import tpu as pltpu
from jax.experimental.pallas import tpu_sc as plsc

NUM_GRAPHS = 1024
OUT_DIM = 1024
NUM_WORKERS = 32
GMAX = 64
CHUNK = 8192
XBUF = CHUNK + 24
PTR_PAD = 1040
SPLIT_PAD = 48
LANES = 16
UNROLL = 4
HIST_WORDS = GMAX * OUT_DIM + 2048

_cp = pltpu.CompilerParams()
if "needs_layout_passes" in pltpu.CompilerParams.__dataclass_fields__:
    _cp = dataclasses.replace(_cp, needs_layout_passes=False)


@functools.partial(
    pl.kernel,
    compiler_params=_cp,
    out_type=jax.ShapeDtypeStruct((NUM_GRAPHS, OUT_DIM), jnp.float32),
    mesh=plsc.VectorSubcoreMesh(core_axis_name="c", subcore_axis_name="s"),
    scratch_types=[
        pltpu.VMEM((PTR_PAD,), jnp.int32),
        pltpu.VMEM((SPLIT_PAD,), jnp.int32),
        pltpu.VMEM((XBUF,), jnp.int32),
        pltpu.VMEM((XBUF,), jnp.int32),
        pltpu.VMEM((HIST_WORDS,), jnp.float32),
        pltpu.SemaphoreType.DMA,
        pltpu.SemaphoreType.DMA,
        pltpu.SemaphoreType.DMA,
    ],
)
def _count_kernel(x_hbm, ptr_hbm, out_hbm, ptr_v, gsplit_v,
                  buf0, buf1, hist, sem0, sem1, wsem):
    total = x_hbm.shape[0]
    wid = lax.axis_index("s") * 2 + lax.axis_index("c")

    zeros16 = jnp.zeros((LANES,), jnp.float32)
    ones16 = jnp.ones((LANES,), jnp.float32)
    iota16 = lax.iota(jnp.int32, LANES)
    udim = jnp.uint32(OUT_DIM)

    pltpu.sync_copy(ptr_hbm, ptr_v)

    prev_max = jnp.int32(0)
    min_carry = jnp.int32(1 << 30)
    for vi in range(SPLIT_PAD // LANES):
        w = vi * LANES + iota16
        t = (w * total) // NUM_WORKERS
        pos = jnp.zeros((LANES,), jnp.int32)
        for sh in (1024, 512, 256, 128, 64, 32, 16, 8, 4, 2, 1):
            cand = pos + (sh - 1)
            vals = plsc.load_gather(ptr_v, [jnp.minimum(cand, NUM_GRAPHS)])
            ok = (cand < NUM_GRAPHS + 1) & (vals < t)
            pos = jnp.where(ok, pos + sh, pos)
        gdn = jnp.maximum(pos - 1, 0)
        gup = jnp.minimum(pos, NUM_GRAPHS)
        vdn = plsc.load_gather(ptr_v, [gdn])
        vup = plsc.load_gather(ptr_v, [gup])
        pick = jnp.where((t - vdn) <= (vup - t), gdn, gup)
        if vi == 0:
            pick = jnp.where(iota16 == 0, 0, pick)
        if vi == 2:
            pick = jnp.where(iota16 == 0, NUM_GRAPHS, pick)
        mono = jnp.maximum(plsc.cummax(pick), prev_max)
        prev_max = mono[LANES - 1]
        low = jnp.maximum(0, NUM_GRAPHS - GMAX * (NUM_WORKERS - w))
        high = jnp.minimum(NUM_GRAPHS, GMAX * w)
        clipped = jnp.clip(mono, low, high)
        y = clipped - GMAX * w
        pmin = jnp.minimum(0 - plsc.cummax(0 - y), min_carry)
        min_carry = pmin[LANES - 1]
        gsplit_v[pl.ds(vi * LANES, LANES)] = GMAX * w + pmin

    gv = gsplit_v[pl.ds(wid, LANES)]
    gs = gv[0]
    ge = gv[1]
    ng = ge - gs

    @plsc.parallel_loop(0, ng * OUT_DIM, step=LANES, unroll=8)
    def _(i):
        hist[pl.ds(i, LANES)] = zeros16

    pw = ptr_v[pl.ds(gs, LANES)]
    wstart = pw[0]
    pe = ptr_v[pl.ds(ge, LANES)]
    wend = pe[0]
    wn = wend - wstart
    base_a = (wstart // 8) * 8
    nch = (wn + (CHUNK - 1)) // CHUNK

    def dma_start(c, buf, sem):
        a = jnp.minimum(base_a + c * CHUNK, total - (CHUNK + 8))
        pltpu.async_copy(x_hbm.at[pl.ds(a, CHUNK + 8)],
                         buf.at[pl.ds(0, CHUNK + 8)], sem)

    def dma_wait(buf, sem):
        pltpu.make_async_copy(x_hbm.at[pl.ds(0, CHUNK + 8)],
                              buf.at[pl.ds(0, CHUNK + 8)], sem).wait()

    def scat(colors, mask, bidx):
        idx = bidx + colors
        plsc.addupdate_scatter(hist, [idx], ones16, mask=mask)

    def process(c, buf, g):
        cs = wstart + c * CHUNK
        a = jnp.minimum(base_a + c * CHUNK, total - (CHUNK + 8))
        off = cs - a
        npc = jnp.minimum(wn - c * CHUNK, CHUNK)
        ce = cs + npc

        def piece_cond(st):
            p, _ = st
            return p < ce

        def piece(st):
            p, g = st
            pv = ptr_v[pl.ds(gs + g + 1, LANES)]
            gend = pv[0]
            e = jnp.minimum(gend, ce)
            n = e - p
            bidx = g * OUT_DIM
            boff = off + (p - cs)
            nfull = n // LANES

            @plsc.parallel_loop(0, nfull, unroll=UNROLL)
            def _(v):
                colors = buf[pl.ds(boff + v * LANES, LANES)]
                mask = plsc.bitcast(colors, jnp.uint32) < udim
                scat(colors, mask, bidx)

            rem = n - nfull * LANES

            @pl.when(rem > 0)
            def _():
                colors = buf[pl.ds(boff + nfull * LANES, LANES)]
                mask = (plsc.bitcast(colors, jnp.uint32) < udim) & (iota16 < rem)
                scat(colors, mask, bidx)

            g = jnp.where(gend <= ce, g + 1, g)
            return (e, g)

        _, g = lax.while_loop(piece_cond, piece, (cs, g))
        return g

    @pl.when(nch > 0)
    def _():
        dma_start(jnp.int32(0), buf0, sem0)

    def pair(i, g):
        c = 2 * i

        @pl.when(c + 1 < nch)
        def _():
            dma_start(c + 1, buf1, sem1)

        dma_wait(buf0, sem0)
        g = process(c, buf0, g)

        def second(g):
            @pl.when(c + 2 < nch)
            def _():
                dma_start(c + 2, buf0, sem0)

            dma_wait(buf1, sem1)
            return process(c + 1, buf1, g)

        return lax.cond(c + 1 < nch, second, lambda g: g, g)

    lax.fori_loop(0, (nch + 1) // 2, pair, jnp.int32(0))

    @pl.loop(0, ng)
    def _(g):
        pltpu.async_copy(hist.at[pl.ds(g * OUT_DIM, OUT_DIM)],
                         out_hbm.at[gs + g], wsem)

    @pl.loop(0, ng)
    def _(g):
        pltpu.make_async_copy(hist.at[pl.ds(0, OUT_DIM)],
                              out_hbm.at[0], wsem).wait()


def kernel(x, ptr):
    x32 = x.astype(jnp.int32)
    ptr32 = ptr.astype(jnp.int32)
    total = x32.shape[0]
    ptrp = jnp.concatenate(
        [ptr32, jnp.full((PTR_PAD - ptr32.shape[0],), total, jnp.int32)])
    return _count_kernel(x32, ptrp)


# speedup vs baseline: 2.2907x; 1.0607x over previous
import dataclasses
import functools

import jax
import jax.numpy as jnp
from jax import lax
from jax.experimental import pallas as pl
from jax.experimental.pallas import tpu as pltpu
from jax.experimental.pallas import tpu_sc as plsc

NUM_GRAPHS = 1024
OUT_DIM = 1024
NUM_WORKERS = 32
GMAX = 64
CHUNK = 16384
XBUF = CHUNK + 24
PTR_PAD = 1040
SPLIT_PAD = 48
LANES = 16
UNROLL = 4
HIST_WORDS = GMAX * OUT_DIM + 2048

_cp = pltpu.CompilerParams()
if "needs_layout_passes" in pltpu.CompilerParams.__dataclass_fields__:
    _cp = dataclasses.replace(_cp, needs_layout_passes=False)


@functools.partial(
    pl.kernel,
    compiler_params=_cp,
    out_type=jax.ShapeDtypeStruct((NUM_GRAPHS, OUT_DIM), jnp.float32),
    mesh=plsc.VectorSubcoreMesh(core_axis_name="c", subcore_axis_name="s"),
    scratch_types=[
        pltpu.VMEM((PTR_PAD,), jnp.int32),
        pltpu.VMEM((SPLIT_PAD,), jnp.int32),
        pltpu.VMEM((XBUF,), jnp.int32),
        pltpu.VMEM((XBUF,), jnp.int32),
        pltpu.VMEM((HIST_WORDS,), jnp.float32),
        pltpu.SemaphoreType.DMA,
        pltpu.SemaphoreType.DMA,
        pltpu.SemaphoreType.DMA,
    ],
)
def _count_kernel(x_hbm, ptr_hbm, out_hbm, ptr_v, gsplit_v,
                  buf0, buf1, hist, sem0, sem1, wsem):
    total = x_hbm.shape[0]
    wid = lax.axis_index("s") * 2 + lax.axis_index("c")

    zeros16 = jnp.zeros((LANES,), jnp.float32)
    ones16 = jnp.ones((LANES,), jnp.float32)
    iota16 = lax.iota(jnp.int32, LANES)
    udim = jnp.uint32(OUT_DIM)

    pltpu.sync_copy(ptr_hbm, ptr_v.at[pl.ds(0, NUM_GRAPHS + 1)])

    prev_max = jnp.int32(0)
    min_carry = jnp.int32(1 << 30)
    for vi in range(SPLIT_PAD // LANES):
        w = vi * LANES + iota16
        t = (w * total) // NUM_WORKERS
        pos = jnp.zeros((LANES,), jnp.int32)
        for sh in (1024, 512, 256, 128, 64, 32, 16, 8, 4, 2, 1):
            cand = pos + (sh - 1)
            vals = plsc.load_gather(ptr_v, [jnp.minimum(cand, NUM_GRAPHS)])
            ok = (cand < NUM_GRAPHS + 1) & (vals < t)
            pos = jnp.where(ok, pos + sh, pos)
        gdn = jnp.maximum(pos - 1, 0)
        gup = jnp.minimum(pos, NUM_GRAPHS)
        vdn = plsc.load_gather(ptr_v, [gdn])
        vup = plsc.load_gather(ptr_v, [gup])
        pick = jnp.where((t - vdn) <= (vup - t), gdn, gup)
        if vi == 0:
            pick = jnp.where(iota16 == 0, 0, pick)
        if vi == 2:
            pick = jnp.where(iota16 == 0, NUM_GRAPHS, pick)
        mono = jnp.maximum(plsc.cummax(pick), prev_max)
        prev_max = mono[LANES - 1]
        low = jnp.maximum(0, NUM_GRAPHS - GMAX * (NUM_WORKERS - w))
        high = jnp.minimum(NUM_GRAPHS, GMAX * w)
        clipped = jnp.clip(mono, low, high)
        y = clipped - GMAX * w
        pmin = jnp.minimum(0 - plsc.cummax(0 - y), min_carry)
        min_carry = pmin[LANES - 1]
        gsplit_v[pl.ds(vi * LANES, LANES)] = GMAX * w + pmin

    gv = gsplit_v[pl.ds(wid, LANES)]
    gs = gv[0]
    ge = gv[1]
    ng = ge - gs

    @plsc.parallel_loop(0, ng * OUT_DIM, step=LANES, unroll=8)
    def _(i):
        hist[pl.ds(i, LANES)] = zeros16

    pw = ptr_v[pl.ds(gs, LANES)]
    wstart = pw[0]
    pe = ptr_v[pl.ds(ge, LANES)]
    wend = pe[0]
    wn = wend - wstart
    base_a = (wstart // 8) * 8
    nch = (wn + (CHUNK - 1)) // CHUNK

    def dma_start(c, buf, sem):
        a = jnp.minimum(base_a + c * CHUNK, total - (CHUNK + 8))
        pltpu.async_copy(x_hbm.at[pl.ds(a, CHUNK + 8)],
                         buf.at[pl.ds(0, CHUNK + 8)], sem)

    def dma_wait(buf, sem):
        pltpu.make_async_copy(x_hbm.at[pl.ds(0, CHUNK + 8)],
                              buf.at[pl.ds(0, CHUNK + 8)], sem).wait()

    def scat(colors, mask, bidx):
        idx = bidx + colors
        plsc.addupdate_scatter(hist, [idx], ones16, mask=mask)

    def process(c, buf, g):
        cs = wstart + c * CHUNK
        a = jnp.minimum(base_a + c * CHUNK, total - (CHUNK + 8))
        off = cs - a
        npc = jnp.minimum(wn - c * CHUNK, CHUNK)
        ce = cs + npc

        def piece_cond(st):
            p, _ = st
            return p < ce

        def piece(st):
            p, g = st
            pv = ptr_v[pl.ds(gs + g + 1, LANES)]
            gend = pv[0]
            e = jnp.minimum(gend, ce)
            n = e - p
            bidx = g * OUT_DIM
            boff = off + (p - cs)
            nfull = n // LANES

            @plsc.parallel_loop(0, nfull, unroll=UNROLL)
            def _(v):
                colors = buf[pl.ds(boff + v * LANES, LANES)]
                mask = plsc.bitcast(colors, jnp.uint32) < udim
                scat(colors, mask, bidx)

            rem = n - nfull * LANES

            @pl.when(rem > 0)
            def _():
                colors = buf[pl.ds(boff + nfull * LANES, LANES)]
                mask = (plsc.bitcast(colors, jnp.uint32) < udim) & (iota16 < rem)
                scat(colors, mask, bidx)

            @pl.when(gend <= ce)
            def _():
                pltpu.async_copy(hist.at[pl.ds(g * OUT_DIM, OUT_DIM)],
                                 out_hbm.at[gs + g], wsem)

            g = jnp.where(gend <= ce, g + 1, g)
            return (e, g)

        _, g = lax.while_loop(piece_cond, piece, (cs, g))
        return g

    @pl.when(nch > 0)
    def _():
        dma_start(jnp.int32(0), buf0, sem0)

    def pair(i, g):
        c = 2 * i

        @pl.when(c + 1 < nch)
        def _():
            dma_start(c + 1, buf1, sem1)

        dma_wait(buf0, sem0)
        g = process(c, buf0, g)

        def second(g):
            @pl.when(c + 2 < nch)
            def _():
                dma_start(c + 2, buf0, sem0)

            dma_wait(buf1, sem1)
            return process(c + 1, buf1, g)

        return lax.cond(c + 1 < nch, second, lambda g: g, g)

    lax.fori_loop(0, (nch + 1) // 2, pair, jnp.int32(0))

    @pl.loop(0, ng)
    def _(g):
        pltpu.make_async_copy(hist.at[pl.ds(0, OUT_DIM)],
                              out_hbm.at[0], wsem).wait()


def kernel(x, ptr):
    x32 = x.astype(jnp.int32)
    ptr32 = ptr.astype(jnp.int32)
    return _count_kernel(x32, ptr32)


# speedup vs baseline: 2.3377x; 1.0205x over previous
import dataclasses
import functools

import jax
import jax.numpy as jnp
from jax import lax
from jax.experimental import pallas as pl
from jax.experimental.pallas import tpu as pltpu
from jax.experimental.pallas import tpu_sc as plsc

NUM_GRAPHS = 1024
OUT_DIM = 1024
NUM_WORKERS = 32
GMAX = 64
CHUNK = 16384
XBUF = CHUNK + 24
PTR_PAD = 1040
SPLIT_PAD = 48
LANES = 16
UNROLL = 4
HIST_WORDS = GMAX * OUT_DIM + 2048

_cp = pltpu.CompilerParams()
if "needs_layout_passes" in pltpu.CompilerParams.__dataclass_fields__:
    _cp = dataclasses.replace(_cp, needs_layout_passes=False)


@functools.partial(
    pl.kernel,
    compiler_params=_cp,
    out_type=jax.ShapeDtypeStruct((NUM_GRAPHS, OUT_DIM), jnp.float32),
    mesh=plsc.VectorSubcoreMesh(core_axis_name="c", subcore_axis_name="s"),
    scratch_types=[
        pltpu.VMEM((PTR_PAD,), jnp.int32),
        pltpu.VMEM((SPLIT_PAD,), jnp.int32),
        pltpu.VMEM((XBUF,), jnp.int32),
        pltpu.VMEM((XBUF,), jnp.int32),
        pltpu.VMEM((HIST_WORDS,), jnp.float32),
        pltpu.SemaphoreType.DMA,
        pltpu.SemaphoreType.DMA,
        pltpu.SemaphoreType.DMA,
    ],
)
def _count_kernel(x_hbm, ptr_hbm, out_hbm, ptr_v, gsplit_v,
                  buf0, buf1, hist, sem0, sem1, wsem):
    total = x_hbm.shape[0]
    wid = lax.axis_index("s") * 2 + lax.axis_index("c")

    zeros16 = jnp.zeros((LANES,), jnp.float32)
    ones16 = jnp.ones((LANES,), jnp.float32)
    iota16 = lax.iota(jnp.int32, LANES)
    udim = jnp.uint32(OUT_DIM)

    pltpu.sync_copy(ptr_hbm, ptr_v.at[pl.ds(0, NUM_GRAPHS + 1)])

    prev_max = jnp.int32(0)
    min_carry = jnp.int32(1 << 30)
    for vi in range(SPLIT_PAD // LANES):
        w = vi * LANES + iota16
        t = (w * total) // NUM_WORKERS
        pos = jnp.zeros((LANES,), jnp.int32)
        for sh in (1024, 512, 256, 128, 64, 32, 16, 8, 4, 2, 1):
            cand = pos + (sh - 1)
            vals = plsc.load_gather(ptr_v, [jnp.minimum(cand, NUM_GRAPHS)])
            ok = (cand < NUM_GRAPHS + 1) & (vals < t)
            pos = jnp.where(ok, pos + sh, pos)
        gdn = jnp.maximum(pos - 1, 0)
        gup = jnp.minimum(pos, NUM_GRAPHS)
        vdn = plsc.load_gather(ptr_v, [gdn])
        vup = plsc.load_gather(ptr_v, [gup])
        pick = jnp.where((t - vdn) <= (vup - t), gdn, gup)
        if vi == 0:
            pick = jnp.where(iota16 == 0, 0, pick)
        if vi == 2:
            pick = jnp.where(iota16 == 0, NUM_GRAPHS, pick)
        mono = jnp.maximum(plsc.cummax(pick), prev_max)
        prev_max = mono[LANES - 1]
        low = jnp.maximum(0, NUM_GRAPHS - GMAX * (NUM_WORKERS - w))
        high = jnp.minimum(NUM_GRAPHS, GMAX * w)
        clipped = jnp.clip(mono, low, high)
        y = clipped - GMAX * w
        pmin = jnp.minimum(0 - plsc.cummax(0 - y), min_carry)
        min_carry = pmin[LANES - 1]
        gsplit_v[pl.ds(vi * LANES, LANES)] = GMAX * w + pmin

    gv = gsplit_v[pl.ds(wid, LANES)]
    gs = gv[0]
    ge = gv[1]
    ng = ge - gs

    pw = ptr_v[pl.ds(gs, LANES)]
    wstart = pw[0]
    pe = ptr_v[pl.ds(ge, LANES)]
    wend = pe[0]
    wn = wend - wstart
    base_a = (wstart // 8) * 8
    nch = (wn + (CHUNK - 1)) // CHUNK

    def dma_start(c, buf, sem):
        a = jnp.minimum(base_a + c * CHUNK, total - (CHUNK + 8))
        pltpu.async_copy(x_hbm.at[pl.ds(a, CHUNK + 8)],
                         buf.at[pl.ds(0, CHUNK + 8)], sem)

    @pl.when(nch > 0)
    def _():
        dma_start(jnp.int32(0), buf0, sem0)

    @pl.when(nch > 1)
    def _():
        dma_start(jnp.int32(1), buf1, sem1)

    @plsc.parallel_loop(0, ng * OUT_DIM, step=LANES, unroll=8)
    def _(i):
        hist[pl.ds(i, LANES)] = zeros16

    def dma_wait(buf, sem):
        pltpu.make_async_copy(x_hbm.at[pl.ds(0, CHUNK + 8)],
                              buf.at[pl.ds(0, CHUNK + 8)], sem).wait()

    def scat(colors, mask, bidx):
        idx = bidx + colors
        plsc.addupdate_scatter(hist, [idx], ones16, mask=mask)

    def process(c, buf, g):
        cs = wstart + c * CHUNK
        a = jnp.minimum(base_a + c * CHUNK, total - (CHUNK + 8))
        off = cs - a
        npc = jnp.minimum(wn - c * CHUNK, CHUNK)
        ce = cs + npc

        def piece_cond(st):
            p, _ = st
            return p < ce

        def piece(st):
            p, g = st
            pv = ptr_v[pl.ds(gs + g + 1, LANES)]
            gend = pv[0]
            e = jnp.minimum(gend, ce)
            n = e - p
            bidx = g * OUT_DIM
            boff = off + (p - cs)
            nfull = n // LANES

            @plsc.parallel_loop(0, nfull, unroll=UNROLL)
            def _(v):
                colors = buf[pl.ds(boff + v * LANES, LANES)]
                mask = plsc.bitcast(colors, jnp.uint32) < udim
                scat(colors, mask, bidx)

            rem = n - nfull * LANES

            @pl.when(rem > 0)
            def _():
                colors = buf[pl.ds(boff + nfull * LANES, LANES)]
                mask = (plsc.bitcast(colors, jnp.uint32) < udim) & (iota16 < rem)
                scat(colors, mask, bidx)

            @pl.when(gend <= ce)
            def _():
                pltpu.async_copy(hist.at[pl.ds(g * OUT_DIM, OUT_DIM)],
                                 out_hbm.at[gs + g], wsem)

            g = jnp.where(gend <= ce, g + 1, g)
            return (e, g)

        _, g = lax.while_loop(piece_cond, piece, (cs, g))
        return g

    def pair(i, g):
        c = 2 * i

        dma_wait(buf0, sem0)
        g = process(c, buf0, g)

        @pl.when(c + 2 < nch)
        def _():
            dma_start(c + 2, buf0, sem0)

        def second(g):
            dma_wait(buf1, sem1)
            g = process(c + 1, buf1, g)

            @pl.when(c + 3 < nch)
            def _():
                dma_start(c + 3, buf1, sem1)

            return g

        return lax.cond(c + 1 < nch, second, lambda g: g, g)

    lax.fori_loop(0, (nch + 1) // 2, pair, jnp.int32(0))

    @pl.loop(0, ng)
    def _(g):
        pltpu.make_async_copy(hist.at[pl.ds(0, OUT_DIM)],
                              out_hbm.at[0], wsem).wait()


def kernel(x, ptr):
    x32 = x.astype(jnp.int32)
    ptr32 = ptr.astype(jnp.int32)
    return _count_kernel(x32, ptr32)
